# baseline (speedup 1.0000x reference)
import functools

import jax
import jax.numpy as jnp
from jax import lax
from jax.experimental import pallas as pl
from jax.experimental.pallas import tpu as pltpu
from jax.experimental.pallas import tpu_sc as plsc

S = 4
D = 128
NSIG = 3
NLANE = 16
NCOL = D // NLANE
W = S * NSIG

NC = 2
NS = 16
NW = NC * NS

CH = 8
LEVELS = 256


def _build_kernel(b_dim, t_dim):
    n_chunks = t_dim // CH
    assert b_dim == 2 * NW and n_chunks % 4 == 0
    mesh = plsc.VectorSubcoreMesh(core_axis_name="c", subcore_axis_name="s")
    f32 = jnp.float32
    i32 = jnp.int32
    out_sds = jax.ShapeDtypeStruct((b_dim, t_dim, D), f32)
    SGW = CH * W
    SGB = SGW + NLANE
    buf = pltpu.VMEM((2, CH, D), f32)

    @functools.partial(
        pl.kernel,
        mesh=mesh,
        out_type=[out_sds] * 4,
        scratch_types=[
            pltpu.VMEM((3 * LEVELS, D), f32),
            buf, buf, buf, buf,
            pltpu.VMEM((2 * SGB,), i32),
            pltpu.VMEM((2 * SGB,), i32),
            buf, buf,
            buf, buf,
            buf, buf,
            pltpu.SemaphoreType.DMA,
            pltpu.SemaphoreType.DMA,
            pltpu.SemaphoreType.DMA,
        ],
    )
    def sc_kernel(states_h, sig_h, e1_h, e2_h, e3_h,
                  o0_h, o1_h, o2_h, o3_h,
                  et_v, st0, st1, st2, st3,
                  sg0, sg1,
                  o1a, o1b, o2a, o2b, o3a, o3b,
                  sem_in, sem_o0, sem_o1):
        wid = lax.axis_index("s") * NC + lax.axis_index("c")
        b0 = wid * 2
        sem_out = (sem_o0, sem_o1)
        st_l = (st0, st1, st2, st3)
        sg_l = (sg0, sg1)
        o1_l, o2_l, o3_l = (o1a, o1b), (o2a, o2b), (o3a, o3b)

        pltpu.sync_copy(e1_h, et_v.at[pl.ds(0, LEVELS)])
        pltpu.sync_copy(e2_h, et_v.at[pl.ds(LEVELS, LEVELS)])
        pltpu.sync_copy(e3_h, et_v.at[pl.ds(2 * LEVELS, LEVELS)])

        def fire_in(t, p4, p2):
            ti = t * CH
            pltpu.async_copy(states_h.at[pl.ds(b0, 2), pl.ds(ti, CH)],
                             st_l[p4], sem_in)
            for b2 in range(2):
                off = ((b0 + b2) * t_dim + ti) * W
                pltpu.async_copy(sig_h.at[pl.ds(off, SGW)],
                                 sg_l[p2].at[pl.ds(b2 * SGB, SGW)], sem_in)

        def wait_in(p4, p2):
            pltpu.make_async_copy(states_h.at[pl.ds(0, 2), pl.ds(0, CH)],
                                  st_l[p4], sem_in).wait()
            for b2 in range(2):
                pltpu.make_async_copy(sig_h.at[pl.ds(0, SGW)],
                                      sg_l[p2].at[pl.ds(b2 * SGB, SGW)],
                                      sem_in).wait()

        def wait_out(p2):
            for _ in range(4):
                pltpu.make_async_copy(
                    o1_l[p2], o1_h.at[pl.ds(0, 2), pl.ds(0, CH)],
                    sem_out[p2]).wait()

        fire_in(0, 0, 0)

        def outer_body(o, _):
            for p in range(4):
                t = o * 4 + p
                p2 = p % 2

                wait_in(p, p2)

                @pl.when(t >= 2)
                def _():
                    wait_out(p2)

                @pl.when(t + 1 < n_chunks)
                def _():
                    fire_in(t + 1, (p + 1) % 4, (p + 1) % 2)

                sg = sg_l[p2]

                @plsc.parallel_loop(0, 2 * CH, unroll=4)
                def pos_body(j):
                    b2 = lax.shift_right_logical(j, 3)
                    jj = lax.bitwise_and(j, CH - 1)
                    iv = sg[pl.ds(b2 * SGB + jj * W, NLANE)]
                    a, b, c0 = iv[3], iv[4], iv[5]
                    d0, e, f = (iv[6] + LEVELS, iv[7] + LEVELS,
                                iv[8] + LEVELS)
                    g, h, i = (iv[9] + 2 * LEVELS, iv[10] + 2 * LEVELS,
                               iv[11] + 2 * LEVELS)
                    for c in range(NCOL):
                        sl = pl.ds(c * NLANE, NLANE)
                        acc = st_l[p][b2, jj, sl]
                        acc = acc + et_v[a, sl] + et_v[b, sl] + et_v[c0, sl]
                        o1_l[p2][b2, jj, sl] = acc
                        acc = acc + et_v[d0, sl] + et_v[e, sl] + et_v[f, sl]
                        o2_l[p2][b2, jj, sl] = acc
                        acc = acc + et_v[g, sl] + et_v[h, sl] + et_v[i, sl]
                        o3_l[p2][b2, jj, sl] = acc

                ti = t * CH
                db, dt = pl.ds(b0, 2), pl.ds(ti, CH)
                pltpu.async_copy(st_l[p], o0_h.at[db, dt], sem_out[p2])
                pltpu.async_copy(o1_l[p2], o1_h.at[db, dt], sem_out[p2])
                pltpu.async_copy(o2_l[p2], o2_h.at[db, dt], sem_out[p2])
                pltpu.async_copy(o3_l[p2], o3_h.at[db, dt], sem_out[p2])
            return 0

        lax.fori_loop(0, n_chunks // 4, outer_body, 0)
        wait_out(0)
        wait_out(1)

    return sc_kernel


def kernel(states, signals, emb1, emb2, emb3):
    b_dim, t_dim, _ = states.shape
    sig_flat = signals.astype(jnp.int32).reshape(-1)
    return tuple(_build_kernel(b_dim, t_dim)(states, sig_flat, emb1, emb2, emb3))


# speedup vs baseline: 1.1777x; 1.1777x over previous
import functools

import jax
import jax.numpy as jnp
from jax import lax
from jax.experimental import pallas as pl
from jax.experimental.pallas import tpu as pltpu
from jax.experimental.pallas import tpu_sc as plsc

S = 4
D = 128
NSIG = 3
NLANE = 16
NCOL = D // NLANE
W = S * NSIG

NC = 2
NS = 16
NW = NC * NS

CH = 8
LEVELS = 256


def _build_kernel(b_dim, t_dim):
    n_chunks = t_dim // CH
    assert b_dim == 2 * NW and n_chunks % 4 == 0
    mesh = plsc.VectorSubcoreMesh(core_axis_name="c", subcore_axis_name="s")
    f32 = jnp.float32
    i32 = jnp.int32
    out_sds = jax.ShapeDtypeStruct((b_dim, t_dim, D), f32)
    SGW = CH * W
    SGB = SGW + NLANE
    buf = pltpu.VMEM((2, CH, D), f32)

    @functools.partial(
        pl.kernel,
        mesh=mesh,
        out_type=[out_sds] * 4,
        scratch_types=[
            pltpu.VMEM((3 * LEVELS, D), f32),
            buf, buf, buf, buf,
            pltpu.VMEM((2 * SGB,), i32),
            pltpu.VMEM((2 * SGB,), i32),
            buf, buf,
            buf, buf,
            buf, buf,
            pltpu.SemaphoreType.DMA,
            pltpu.SemaphoreType.DMA,
            pltpu.SemaphoreType.DMA,
        ],
    )
    def sc_kernel(states_h, sig_h, e1_h, e2_h, e3_h,
                  o0_h, o1_h, o2_h, o3_h,
                  et_v, st0, st1, st2, st3,
                  sg0, sg1,
                  o1a, o1b, o2a, o2b, o3a, o3b,
                  sem_in, sem_o0, sem_o1):
        wid = lax.axis_index("s") * NC + lax.axis_index("c")
        b0 = wid * 2
        sem_out = (sem_o0, sem_o1)
        st_l = (st0, st1, st2, st3)
        sg_l = (sg0, sg1)
        o1_l, o2_l, o3_l = (o1a, o1b), (o2a, o2b), (o3a, o3b)

        pltpu.sync_copy(e1_h, et_v.at[pl.ds(0, LEVELS)])
        pltpu.sync_copy(e2_h, et_v.at[pl.ds(LEVELS, LEVELS)])
        pltpu.sync_copy(e3_h, et_v.at[pl.ds(2 * LEVELS, LEVELS)])

        def fire_in(t, p4, p2):
            ti = t * CH
            pltpu.async_copy(states_h.at[pl.ds(b0, 2), pl.ds(ti, CH)],
                             st_l[p4], sem_in)
            for b2 in range(2):
                off = ((b0 + b2) * t_dim + ti) * W
                pltpu.async_copy(sig_h.at[pl.ds(off, SGW)],
                                 sg_l[p2].at[pl.ds(b2 * SGB, SGW)], sem_in)

        def wait_in(p4, p2):
            pltpu.make_async_copy(states_h.at[pl.ds(0, 2), pl.ds(0, CH)],
                                  st_l[p4], sem_in).wait()
            for b2 in range(2):
                pltpu.make_async_copy(sig_h.at[pl.ds(0, SGW)],
                                      sg_l[p2].at[pl.ds(b2 * SGB, SGW)],
                                      sem_in).wait()

        def wait_out(p2):
            for _ in range(4):
                pltpu.make_async_copy(
                    o1_l[p2], o1_h.at[pl.ds(0, 2), pl.ds(0, CH)],
                    sem_out[p2]).wait()

        fire_in(0, 0, 0)

        def outer_body(o, _):
            for p in range(4):
                t = o * 4 + p
                p2 = p % 2

                wait_in(p, p2)

                @pl.when(t >= 2)
                def _():
                    wait_out(p2)

                @pl.when(t + 1 < n_chunks)
                def _():
                    fire_in(t + 1, (p + 1) % 4, (p + 1) % 2)

                for b2 in range(2):
                    sg = sg_l[p2]
                    sgo = b2 * SGB

                    @plsc.parallel_loop(0, CH, unroll=4)
                    def pos_body(j):
                        iv = sg[pl.ds(sgo + j * W, NLANE)]
                        a, b, c0 = iv[3], iv[4], iv[5]
                        d0, e, f = (iv[6] + LEVELS, iv[7] + LEVELS,
                                    iv[8] + LEVELS)
                        g, h, i = (iv[9] + 2 * LEVELS, iv[10] + 2 * LEVELS,
                                   iv[11] + 2 * LEVELS)
                        for c in range(NCOL):
                            sl = pl.ds(c * NLANE, NLANE)
                            acc = st_l[p][b2, j, sl]
                            acc = acc + et_v[a, sl] + et_v[b, sl] + et_v[c0, sl]
                            o1_l[p2][b2, j, sl] = acc
                            acc = acc + et_v[d0, sl] + et_v[e, sl] + et_v[f, sl]
                            o2_l[p2][b2, j, sl] = acc
                            acc = acc + et_v[g, sl] + et_v[h, sl] + et_v[i, sl]
                            o3_l[p2][b2, j, sl] = acc

                ti = t * CH
                db, dt = pl.ds(b0, 2), pl.ds(ti, CH)
                pltpu.async_copy(st_l[p], o0_h.at[db, dt], sem_out[p2])
                pltpu.async_copy(o1_l[p2], o1_h.at[db, dt], sem_out[p2])
                pltpu.async_copy(o2_l[p2], o2_h.at[db, dt], sem_out[p2])
                pltpu.async_copy(o3_l[p2], o3_h.at[db, dt], sem_out[p2])
            return 0

        lax.fori_loop(0, n_chunks // 4, outer_body, 0)
        wait_out(0)
        wait_out(1)

    return sc_kernel


def kernel(states, signals, emb1, emb2, emb3):
    b_dim, t_dim, _ = states.shape
    sig_flat = signals.astype(jnp.int32).reshape(-1)
    return tuple(_build_kernel(b_dim, t_dim)(states, sig_flat, emb1, emb2, emb3))
